# Initial kernel scaffold; baseline (speedup 1.0000x reference)
import functools

import jax
import jax.numpy as jnp
from jax.experimental import pallas as pl
from jax.experimental.pallas import tpu as pltpu
from jax.experimental.pallas import tpu_sc as plsc

_D = 64
_W = 128


def _lookup(flat_ids, weight):
    n = flat_ids.shape[0]
    mesh = plsc.VectorSubcoreMesh(core_axis_name="core", subcore_axis_name="subcore")

    @functools.partial(
        pl.kernel,
        out_type=jax.ShapeDtypeStruct((n, _D), weight.dtype),
        mesh=mesh,
    )
    def k(w_hbm, i_hbm, o_hbm):
        def body(i_vmem, o_vmem):
            pltpu.sync_copy(w_hbm.at[i_vmem.at[0]], o_vmem)

        pltpu.emit_pipeline(
            body,
            grid=(n // _W,),
            in_specs=[pl.BlockSpec((1, _W), index_map=lambda i: (0, i))],
            out_specs=[pl.BlockSpec((_W, _D), index_map=lambda i: (i, 0))],
            core_axis_name=("core", "subcore"),
            dimension_semantics=(pltpu.PARALLEL,),
        )(i_hbm, o_hbm)

    return k(weight, flat_ids.reshape(1, n))


def kernel(token_ids, weight):
    batch, hist = token_ids.shape
    out = _lookup(token_ids.reshape(-1), weight)
    return out.reshape(batch, hist, weight.shape[1])


# speedup vs baseline: 1.7478x; 1.7478x over previous
import functools

import jax
import jax.numpy as jnp
from jax.experimental import pallas as pl
from jax.experimental.pallas import tpu as pltpu
from jax.experimental.pallas import tpu_sc as plsc

_D = 64
_W = 128


def _lookup(flat_ids, weight):
    n = flat_ids.shape[0]
    mesh = plsc.VectorSubcoreMesh(core_axis_name="core", subcore_axis_name="subcore")

    @functools.partial(
        pl.kernel,
        out_type=jax.ShapeDtypeStruct((n, _D), weight.dtype),
        mesh=mesh,
        compiler_params=pltpu.CompilerParams(use_tc_tiling_on_sc=False),
    )
    def k(w_hbm, i_hbm, o_hbm):
        def body(i_vmem, o_vmem):
            pltpu.sync_copy(w_hbm.at[i_vmem.at[0]], o_vmem)

        pltpu.emit_pipeline(
            body,
            grid=(n // _W,),
            in_specs=[pl.BlockSpec((1, _W), index_map=lambda i: (0, i))],
            out_specs=[pl.BlockSpec((_W, _D), index_map=lambda i: (i, 0))],
            core_axis_name=("core", "subcore"),
            dimension_semantics=(pltpu.PARALLEL,),
        )(i_hbm, o_hbm)

    return k(weight, flat_ids.reshape(1, n))


def kernel(token_ids, weight):
    batch, hist = token_ids.shape
    out = _lookup(token_ids.reshape(-1), weight)
    return out.reshape(batch, hist, weight.shape[1])


# speedup vs baseline: 1.8694x; 1.0696x over previous
import functools

import jax
import jax.numpy as jnp
from jax.experimental import pallas as pl
from jax.experimental.pallas import tpu as pltpu
from jax.experimental.pallas import tpu_sc as plsc

_D = 64
_W = 128
_K = 4


def _lookup(flat_ids, weight):
    n = flat_ids.shape[0]
    rows_per_step = _K * _W
    mesh = plsc.VectorSubcoreMesh(core_axis_name="core", subcore_axis_name="subcore")

    @functools.partial(
        pl.kernel,
        out_type=jax.ShapeDtypeStruct((n, _D), weight.dtype),
        mesh=mesh,
        scratch_types=[pltpu.SemaphoreType.DMA],
        compiler_params=pltpu.CompilerParams(use_tc_tiling_on_sc=False),
    )
    def k(w_hbm, i_hbm, o_hbm, sem):
        def body(i_vmem, o_vmem):
            copies = [
                pltpu.async_copy(
                    w_hbm.at[i_vmem.at[0, j]],
                    o_vmem.at[pl.ds(j * _W, _W)],
                    sem,
                )
                for j in range(_K)
            ]
            for c in copies:
                c.wait()

        pltpu.emit_pipeline(
            body,
            grid=(n // rows_per_step,),
            in_specs=[pl.BlockSpec((1, _K, _W), index_map=lambda i: (i, 0, 0))],
            out_specs=[pl.BlockSpec((rows_per_step, _D), index_map=lambda i: (i, 0))],
            core_axis_name=("core", "subcore"),
            dimension_semantics=(pltpu.PARALLEL,),
        )(i_hbm, o_hbm)

    return k(weight, flat_ids.reshape(n // rows_per_step, _K, _W))


def kernel(token_ids, weight):
    batch, hist = token_ids.shape
    out = _lookup(token_ids.reshape(-1), weight)
    return out.reshape(batch, hist, weight.shape[1])


# speedup vs baseline: 1.8716x; 1.0012x over previous
import functools

import jax
import jax.numpy as jnp
from jax.experimental import pallas as pl
from jax.experimental.pallas import tpu as pltpu
from jax.experimental.pallas import tpu_sc as plsc

_D = 64
_W = 512
_K = 1


def _lookup(flat_ids, weight):
    n = flat_ids.shape[0]
    rows_per_step = _K * _W
    mesh = plsc.VectorSubcoreMesh(core_axis_name="core", subcore_axis_name="subcore")

    @functools.partial(
        pl.kernel,
        out_type=jax.ShapeDtypeStruct((n, _D), weight.dtype),
        mesh=mesh,
        scratch_types=[pltpu.SemaphoreType.DMA],
        compiler_params=pltpu.CompilerParams(use_tc_tiling_on_sc=False),
    )
    def k(w_hbm, i_hbm, o_hbm, sem):
        def body(i_vmem, o_vmem):
            copies = [
                pltpu.async_copy(
                    w_hbm.at[i_vmem.at[0, j]],
                    o_vmem.at[pl.ds(j * _W, _W)],
                    sem,
                )
                for j in range(_K)
            ]
            for c in copies:
                c.wait()

        pltpu.emit_pipeline(
            body,
            grid=(n // rows_per_step,),
            in_specs=[pl.BlockSpec((1, _K, _W), index_map=lambda i: (i, 0, 0))],
            out_specs=[pl.BlockSpec((rows_per_step, _D), index_map=lambda i: (i, 0))],
            core_axis_name=("core", "subcore"),
            dimension_semantics=(pltpu.PARALLEL,),
        )(i_hbm, o_hbm)

    return k(weight, flat_ids.reshape(n // rows_per_step, _K, _W))


def kernel(token_ids, weight):
    batch, hist = token_ids.shape
    out = _lookup(token_ids.reshape(-1), weight)
    return out.reshape(batch, hist, weight.shape[1])
